# Initial kernel scaffold; baseline (speedup 1.0000x reference)
import jax
import jax.numpy as jnp
from jax.experimental import pallas as pl


def kernel(chunk, context, W_sim, b_sim, Wq, bq, Wk, bk, Wv, bv, Wo, bo):
    raise NotImplementedError("write your pallas kernel here")


# speedup vs baseline: 2.3394x; 2.3394x over previous
import functools

import numpy as np
import jax
import jax.numpy as jnp
from jax import lax
from jax.experimental import pallas as pl
from jax.experimental.pallas import tpu as pltpu
from jax.experimental.pallas import tpu_sc as plsc

_B, _LQ, _LC, _D, _H, _K = 2, 2048, 4096, 1024, 16, 16
_DH = _D // _H
_QTOT = _B * _LQ
_CTOT = _B * _LC
_NC, _NS, _L = 2, 16, 16
_NW = _NC * _NS
_QPW = _QTOT // _NW
_BM = 512
_BQ, _BC = 256, 1024


def _ctx_triple_body(x_ref, wsimbf_ref, bsim_ref, wk_ref, bk_ref,
                     wv_ref, bv_ref, cpb_ref, kc_ref, vc_ref):
    x = x_ref[...]
    dn = (((1,), (1,)), ((), ()))
    cp = lax.dot_general(x.astype(jnp.bfloat16), wsimbf_ref[...], dn,
                         preferred_element_type=jnp.float32) + bsim_ref[...]
    cpb_ref[...] = cp.astype(jnp.bfloat16)
    kc_ref[...] = lax.dot_general(x, wk_ref[...], dn, precision=lax.Precision.HIGHEST,
                                  preferred_element_type=jnp.float32) + bk_ref[...]
    vc_ref[...] = lax.dot_general(x, wv_ref[...], dn, precision=lax.Precision.HIGHEST,
                                  preferred_element_type=jnp.float32) + bv_ref[...]


def _ctx_triple(x, wsim_bf, bsim, wk, bk, wv, bv):
    m = x.shape[0]
    full = lambda i: (0, 0)
    return pl.pallas_call(
        _ctx_triple_body,
        grid=(m // _BM,),
        in_specs=[
            pl.BlockSpec((_BM, _D), lambda i: (i, 0)),
            pl.BlockSpec((_D, _D), full),
            pl.BlockSpec((1, _D), full),
            pl.BlockSpec((_D, _D), full),
            pl.BlockSpec((1, _D), full),
            pl.BlockSpec((_D, _D), full),
            pl.BlockSpec((1, _D), full),
        ],
        out_specs=[
            pl.BlockSpec((_BM, _D), lambda i: (i, 0)),
            pl.BlockSpec((_BM, _D), lambda i: (i, 0)),
            pl.BlockSpec((_BM, _D), lambda i: (i, 0)),
        ],
        out_shape=[
            jax.ShapeDtypeStruct((m, _D), jnp.bfloat16),
            jax.ShapeDtypeStruct((m, _D), jnp.float32),
            jax.ShapeDtypeStruct((m, _D), jnp.float32),
        ],
    )(x, wsim_bf, bsim, wk, bk, wv, bv)


def _single_proj_body(x_ref, w_ref, b_ref, o_ref):
    dn = (((1,), (1,)), ((), ()))
    o_ref[...] = lax.dot_general(x_ref[...], w_ref[...], dn, precision=lax.Precision.HIGHEST,
                                 preferred_element_type=jnp.float32) + b_ref[...]


def _single_proj(x, w, b):
    m = x.shape[0]
    return pl.pallas_call(
        _single_proj_body,
        grid=(m // _BM,),
        in_specs=[
            pl.BlockSpec((_BM, _D), lambda i: (i, 0)),
            pl.BlockSpec((_D, _D), lambda i: (0, 0)),
            pl.BlockSpec((1, _D), lambda i: (0, 0)),
        ],
        out_specs=pl.BlockSpec((_BM, _D), lambda i: (i, 0)),
        out_shape=jax.ShapeDtypeStruct((m, _D), jnp.float32),
    )(x, w, b)


def _simtopk_body(cb_ref, cpb_ref, idx_ref, sim_ref):
    b = pl.program_id(0)
    ci = pl.program_id(2)
    s = lax.dot_general(cb_ref[0], cpb_ref[0], (((1,), (1,)), ((), ())),
                        preferred_element_type=jnp.float32)
    sim_ref[:, pl.ds(ci * _BC, _BC)] = s

    @pl.when(ci == _LC // _BC - 1)
    def _():
        vals = sim_ref[...]
        iota = lax.broadcasted_iota(jnp.int32, (_BQ, _LC), 1)
        neg = jnp.float32(-3.0e38)
        cols = []
        for _t in range(_K):
            m = jnp.max(vals, axis=1, keepdims=True)
            it = jnp.min(jnp.where(vals >= m, iota, _LC), axis=1, keepdims=True)
            cols.append(it)
            vals = jnp.where(iota == it, neg, vals)
        idx_ref[0] = jnp.concatenate(cols, axis=1) + b * _LC


def _simtopk(chunk_bf, ctxp_bf):
    return pl.pallas_call(
        _simtopk_body,
        grid=(_B, _LQ // _BQ, _LC // _BC),
        in_specs=[
            pl.BlockSpec((1, _BQ, _D), lambda b, qi, ci: (b, qi, 0)),
            pl.BlockSpec((1, _BC, _D), lambda b, qi, ci: (b, ci, 0)),
        ],
        out_specs=pl.BlockSpec((1, _BQ, _K), lambda b, qi, ci: (b, qi, 0)),
        out_shape=jax.ShapeDtypeStruct((_B, _LQ, _K), jnp.int32),
        scratch_shapes=[pltpu.VMEM((_BQ, _LC), jnp.float32)],
    )(chunk_bf, ctxp_bf)


def _sc_attention_body(q_hbm, kc_hbm, vc_hbm, idx_hbm, out_hbm,
                       idx_v, kg, vg, qv, attn_v, ov, sem_k, sem_v):
    wid = lax.axis_index("s") * _NC + lax.axis_index("c")
    base = wid * _QPW
    lanes = lax.iota(jnp.int32, _L)
    inv_sqrt_dh = jnp.float32(1.0 / np.sqrt(_DH))

    def qbody(i, carry):
        row = base + i
        pltpu.sync_copy(idx_hbm.at[row], idx_v)
        pltpu.sync_copy(q_hbm.at[row], qv)
        ck = pltpu.async_copy(kc_hbm.at[idx_v], kg, sem_k)
        cv = pltpu.async_copy(vc_hbm.at[idx_v], vg, sem_v)
        ck.wait()
        cv.wait()

        for h in range(_H):
            def sbody(j, acc):
                d = h * _DH + j
                dsplat = jnp.full((_L,), d, jnp.int32)
                col = plsc.load_gather(kg, [lanes, dsplat])
                qs = plsc.load_gather(qv, [dsplat])
                return acc + qs * col

            acc = lax.fori_loop(0, _DH, sbody, jnp.zeros((_L,), jnp.float32),
                                unroll=8)
            sc = acc * inv_sqrt_dh
            mx = jnp.max(sc)
            e = jnp.exp(sc - mx)
            attn_v[pl.ds(h * _L, _L)] = e / jnp.sum(e)

        def obody(j, carry2):
            hh = j // (_DH // _L)
            acc = jnp.zeros((_L,), jnp.float32)
            for k in range(_K):
                asplat = plsc.load_gather(
                    attn_v, [jnp.full((_L,), hh * _L + k, jnp.int32)])
                acc = acc + asplat * vg[k, pl.ds(j * _L, _L)]
            ov[pl.ds(j * _L, _L)] = acc
            return carry2

        lax.fori_loop(0, _D // _L, obody, 0, unroll=2)
        pltpu.sync_copy(ov, out_hbm.at[row])
        return carry

    lax.fori_loop(0, _QPW, qbody, 0)


@functools.lru_cache(maxsize=1)
def _sc_attention_kernel():
    mesh = plsc.VectorSubcoreMesh(core_axis_name="c", subcore_axis_name="s")
    return pl.kernel(
        _sc_attention_body,
        mesh=mesh,
        compiler_params=pltpu.CompilerParams(use_tc_tiling_on_sc=False,
                                             needs_layout_passes=False),
        out_type=jax.ShapeDtypeStruct((_QTOT, _D), jnp.float32),
        scratch_types=[
            pltpu.VMEM((_K,), jnp.int32),
            pltpu.VMEM((_K, _D), jnp.float32),
            pltpu.VMEM((_K, _D), jnp.float32),
            pltpu.VMEM((_D,), jnp.float32),
            pltpu.VMEM((_H * _L,), jnp.float32),
            pltpu.VMEM((_D,), jnp.float32),
            pltpu.SemaphoreType.DMA,
            pltpu.SemaphoreType.DMA,
        ],
    )


def kernel(chunk, context, W_sim, b_sim, Wq, bq, Wk, bk, Wv, bv, Wo, bo):
    x_chunk = chunk.reshape(_QTOT, _D)
    x_ctx = context.reshape(_CTOT, _D)

    q = _single_proj(x_chunk, Wq, bq.reshape(1, _D))
    cpb, kc, vc = _ctx_triple(x_ctx, W_sim.astype(jnp.bfloat16),
                              b_sim.reshape(1, _D), Wk, bk.reshape(1, _D),
                              Wv, bv.reshape(1, _D))
    chunk_bf = x_chunk.astype(jnp.bfloat16).reshape(_B, _LQ, _D)
    idx = _simtopk(chunk_bf, cpb.reshape(_B, _LC, _D))
    attn_out = _sc_attention_kernel()(q, kc, vc, idx.reshape(_QTOT, _K))
    out = _single_proj(attn_out, Wo, bo.reshape(1, _D))
    return out.reshape(_B, _LQ, _D)
